# baseline (speedup 1.0000x reference)
import functools

import jax
import jax.numpy as jnp
from jax import lax
from jax.experimental import pallas as pl
from jax.experimental.pallas import tpu as pltpu
from jax.experimental.pallas import tpu_sc as plsc

_TOPK = 8
_NEG = float("-inf")

_B = 128
_N = 32768
_NC = 2
_NS = 16
_NW = _NC * _NS
_RPW = _B // _NW
_CHUNK = 16384
_NCHUNK = _N // _CHUNK
_GV = 16
_GROUPS = _CHUNK // (16 * _GV)
_OPAD = 16

_DNUMS = lax.GatherDimensionNumbers(
    offset_dims=(), collapsed_slice_dims=(0,), start_index_map=(0,))


def _softmax_block(w_ref, o_ref):
    w = w_ref[...]
    m = jnp.max(w, axis=1, keepdims=True)
    e = jnp.exp(w - m)
    o_ref[...] = e / jnp.sum(e, axis=1, keepdims=True)


def _gather16(v, idx16):
    return lax.gather(v, idx16, _DNUMS, (1,),
                      mode=lax.GatherScatterMode.PROMISE_IN_BOUNDS)


def _any16(m):
    return plsc.all_reduce_population_count(m)[0] != 0


def _make_group_body(xb, mwb, chunk_base, dec):
    l_iota, bitsel, widx = dec

    def body(g, carry):
        bv, bi, thr = carry
        base = g * (16 * _GV)

        hits = []
        for blk in range(_GV // 4):
            w = mwb[pl.ds((g * (_GV // 4) + blk) * 16, 16)]
            h = jnp.zeros((16,), jnp.bool_)
            for j in range(4):
                v = xb[pl.ds(base + blk * 64 + j * 16, 16)]
                valid = (_gather16(w, widx[j]) & bitsel) != 0
                h = h | (valid & (v > thr))
            hits.append(h)
        anyhit = hits[0]
        for h in hits[1:]:
            anyhit = anyhit | h

        def merge(c):
            bv, bi, _ = c
            for blk in range(_GV // 4):
                def m_blk(c2, blk=blk):
                    bv, bi = c2
                    w = mwb[pl.ds((g * (_GV // 4) + blk) * 16, 16)]
                    for j in range(4):
                        off = base + blk * 64 + j * 16
                        v = xb[pl.ds(off, 16)]
                        valid = (_gather16(w, widx[j]) & bitsel) != 0
                        mv = jnp.where(valid, v, _NEG)
                        gi = (chunk_base + off) + l_iota
                        sk, si = plsc.sort_key_val(mv, gi, descending=True)
                        p = bv >= sk
                        bv, bi = plsc.sort_key_val(
                            jnp.where(p, bv, sk), jnp.where(p, bi, si))
                    return bv, bi
                bv, bi = lax.cond(_any16(hits[blk]), m_blk,
                                  lambda c2: c2, (bv, bi))
            nthr = _gather16(bv, jnp.full((16, 1), _TOPK, jnp.int32))
            return bv, bi, nthr

        return lax.cond(_any16(anyhit), merge, lambda c: c, (bv, bi, thr))

    return body


_mesh = plsc.VectorSubcoreMesh(core_axis_name="c", subcore_axis_name="s")


@functools.partial(
    pl.kernel,
    mesh=_mesh,
    compiler_params=pltpu.CompilerParams(needs_layout_passes=False),
    out_type=jax.ShapeDtypeStruct((_NW, _OPAD), jnp.float32),
    scratch_types=[
        pltpu.VMEM((_CHUNK,), jnp.float32),
        pltpu.VMEM((_CHUNK,), jnp.float32),
        pltpu.VMEM((_CHUNK // 4,), jnp.int32),
        pltpu.VMEM((_CHUNK // 4,), jnp.int32),
        pltpu.VMEM((16,), jnp.float32),
        pltpu.VMEM((_OPAD,), jnp.float32),
        pltpu.SemaphoreType.DMA,
        pltpu.SemaphoreType.DMA,
        pltpu.SemaphoreType.DMA,
    ],
)
def _sc_wta(x_hbm, mw_hbm, sw_hbm, out_hbm,
            xb0, xb1, mwb0, mwb1, gwb, obuf, sem0, sem1, semw):
    wid = lax.axis_index("s") * _NC + lax.axis_index("c")
    slots = ((xb0, mwb0, sem0), (xb1, mwb1, sem1))

    def start(k):
        r, c = divmod(k, _NCHUNK)
        row = wid * _RPW + r
        xb, mwb, sem = slots[k % 2]
        cx = pltpu.async_copy(
            x_hbm.at[row, pl.ds(c * _CHUNK, _CHUNK)], xb, sem)
        cm = pltpu.async_copy(
            mw_hbm.at[row, pl.ds(c * (_CHUNK // 4), _CHUNK // 4)], mwb, sem)
        return cx, cm

    pend = start(0)

    lane = lax.iota(jnp.int32, 16)
    top = lane >= _TOPK
    widx = [((lane >> 2) + 4 * j)[:, None] for j in range(4)]
    bitsel = lax.shift_left(jnp.int32(1), 8 * (lane & 3))
    dec = (lane, bitsel, widx)

    nk = _RPW * _NCHUNK
    out_vec = jnp.zeros((16,), jnp.float32)
    for r in range(_RPW):
        bv = jnp.full((16,), _NEG, jnp.float32)
        bi = jnp.zeros((16,), jnp.int32)
        thr = jnp.full((16,), _NEG, jnp.float32)
        for c in range(_NCHUNK):
            k = r * _NCHUNK + c
            for cp in pend:
                cp.wait()
            if k + 1 < nk:
                nxt = start(k + 1)
            xb, mwb, _ = slots[k % 2]
            bv, bi, thr = lax.fori_loop(
                0, _GROUPS, _make_group_body(xb, mwb, c * _CHUNK, dec),
                (bv, bi, thr))
            if k + 1 < nk:
                pend = nxt
        pltpu.async_copy(sw_hbm.at[bi], gwb, semw).wait()
        gw = gwb[...]
        s_vw = jnp.sum(jnp.where(top, bv * gw, 0.0))
        s_v = jnp.sum(jnp.where(top, bv, 0.0))
        out_vec = jnp.where(lane == r, s_vw + s_v * (1.0 / _TOPK), out_vec)

    obuf[...] = out_vec
    pltpu.sync_copy(obuf, out_hbm.at[wid])


def kernel(x, mask, W):
    B, N = x.shape
    softw = pl.pallas_call(
        _softmax_block,
        out_shape=jax.ShapeDtypeStruct((1, N), jnp.float32),
    )(W.reshape(1, N))
    mw = lax.bitcast_convert_type(
        mask.view(jnp.uint8).reshape(B, N // 4, 4), jnp.int32)
    out = _sc_wta(x, mw, softw.reshape(N))
    return out[:, :_RPW].reshape(B, 1)


# speedup vs baseline: 1.6044x; 1.6044x over previous
import functools

import jax
import jax.numpy as jnp
from jax import lax
from jax.experimental import pallas as pl
from jax.experimental.pallas import tpu as pltpu
from jax.experimental.pallas import tpu_sc as plsc

_TOPK = 8
_NEG = float("-inf")

_B = 128
_N = 32768
_NC = 2
_NS = 16
_NW = _NC * _NS
_RPW = _B // _NW
_CHUNK = 16384
_NCHUNK = _N // _CHUNK
_GV = 16
_GROUPS = _CHUNK // (16 * _GV)
_OPAD = 16

_DNUMS = lax.GatherDimensionNumbers(
    offset_dims=(), collapsed_slice_dims=(0,), start_index_map=(0,))


def _softmax_block(w_ref, o_ref):
    w = w_ref[...]
    m = jnp.max(w, axis=1, keepdims=True)
    e = jnp.exp(w - m)
    o_ref[...] = e / jnp.sum(e, axis=1, keepdims=True)


def _gather16(v, idx16):
    return lax.gather(v, idx16, _DNUMS, (1,),
                      mode=lax.GatherScatterMode.PROMISE_IN_BOUNDS)


def _any16(m):
    return plsc.all_reduce_population_count(m)[0] != 0


def _make_group_body(xb, mwb, chunk_base, dec):
    (l_iota,) = dec

    def body(g, carry):
        bv, bi, thr = carry
        base = g * (16 * _GV)

        hits = []
        for blk in range(_GV // 4):
            h = jnp.zeros((16,), jnp.bool_)
            for j in range(4):
                off = base + blk * 64 + j * 16
                v = xb[pl.ds(off, 16)]
                valid = mwb[pl.ds(off, 16)] != 0.0
                h = h | (valid & (v > thr))
            hits.append(h)
        anyhit = hits[0]
        for h in hits[1:]:
            anyhit = anyhit | h

        def merge(c):
            bv, bi, _ = c
            for blk in range(_GV // 4):
                def m_blk(c2, blk=blk):
                    bv, bi = c2
                    for j in range(4):
                        off = base + blk * 64 + j * 16
                        v = xb[pl.ds(off, 16)]
                        valid = mwb[pl.ds(off, 16)] != 0.0
                        mv = jnp.where(valid, v, _NEG)
                        gi = (chunk_base + off) + l_iota
                        sk, si = plsc.sort_key_val(mv, gi, descending=True)
                        p = bv >= sk
                        bv, bi = plsc.sort_key_val(
                            jnp.where(p, bv, sk), jnp.where(p, bi, si))
                    return bv, bi
                bv, bi = lax.cond(_any16(hits[blk]), m_blk,
                                  lambda c2: c2, (bv, bi))
            nthr = _gather16(bv, jnp.full((16, 1), _TOPK, jnp.int32))
            return bv, bi, nthr

        return lax.cond(_any16(anyhit), merge, lambda c: c, (bv, bi, thr))

    return body


_mesh = plsc.VectorSubcoreMesh(core_axis_name="c", subcore_axis_name="s")


@functools.partial(
    pl.kernel,
    mesh=_mesh,
    compiler_params=pltpu.CompilerParams(needs_layout_passes=False),
    out_type=jax.ShapeDtypeStruct((_NW, _OPAD), jnp.float32),
    scratch_types=[
        pltpu.VMEM((_CHUNK,), jnp.float32),
        pltpu.VMEM((_CHUNK,), jnp.float32),
        pltpu.VMEM((_CHUNK,), jnp.float32),
        pltpu.VMEM((_CHUNK,), jnp.float32),
        pltpu.VMEM((16,), jnp.float32),
        pltpu.VMEM((_OPAD,), jnp.float32),
        pltpu.SemaphoreType.DMA,
        pltpu.SemaphoreType.DMA,
        pltpu.SemaphoreType.DMA,
    ],
)
def _sc_wta(x_hbm, mw_hbm, sw_hbm, out_hbm,
            xb0, xb1, mwb0, mwb1, gwb, obuf, sem0, sem1, semw):
    wid = lax.axis_index("s") * _NC + lax.axis_index("c")
    slots = ((xb0, mwb0, sem0), (xb1, mwb1, sem1))

    def start(k):
        r, c = divmod(k, _NCHUNK)
        row = wid * _RPW + r
        xb, mwb, sem = slots[k % 2]
        cx = pltpu.async_copy(
            x_hbm.at[row, pl.ds(c * _CHUNK, _CHUNK)], xb, sem)
        cm = pltpu.async_copy(
            mw_hbm.at[row, pl.ds(c * _CHUNK, _CHUNK)], mwb, sem)
        return cx, cm

    pend = start(0)

    lane = lax.iota(jnp.int32, 16)
    top = lane >= _TOPK
    dec = (lane,)

    nk = _RPW * _NCHUNK
    out_vec = jnp.zeros((16,), jnp.float32)
    for r in range(_RPW):
        bv = jnp.full((16,), _NEG, jnp.float32)
        bi = jnp.zeros((16,), jnp.int32)
        thr = jnp.full((16,), _NEG, jnp.float32)
        for c in range(_NCHUNK):
            k = r * _NCHUNK + c
            for cp in pend:
                cp.wait()
            if k + 1 < nk:
                nxt = start(k + 1)
            xb, mwb, _ = slots[k % 2]
            bv, bi, thr = lax.fori_loop(
                0, _GROUPS, _make_group_body(xb, mwb, c * _CHUNK, dec),
                (bv, bi, thr))
            if k + 1 < nk:
                pend = nxt
        pltpu.async_copy(sw_hbm.at[bi], gwb, semw).wait()
        gw = gwb[...]
        s_vw = jnp.sum(jnp.where(top, bv * gw, 0.0))
        s_v = jnp.sum(jnp.where(top, bv, 0.0))
        out_vec = jnp.where(lane == r, s_vw + s_v * (1.0 / _TOPK), out_vec)

    obuf[...] = out_vec
    pltpu.sync_copy(obuf, out_hbm.at[wid])


def kernel(x, mask, W):
    B, N = x.shape
    softw = pl.pallas_call(
        _softmax_block,
        out_shape=jax.ShapeDtypeStruct((1, N), jnp.float32),
    )(W.reshape(1, N))
    mf = mask.astype(jnp.float32)
    out = _sc_wta(x, mf, softw.reshape(N))
    return out[:, :_RPW].reshape(B, 1)
